# Initial kernel scaffold; baseline (speedup 1.0000x reference)
import jax
import jax.numpy as jnp
from jax.experimental import pallas as pl


def kernel(inputs, word_embedding, position_embedding):
    raise NotImplementedError("write your pallas kernel here")


# speedup vs baseline: 3.0641x; 3.0641x over previous
import jax
import jax.numpy as jnp
from jax import lax
from jax.experimental import pallas as pl
from jax.experimental.pallas import tpu as pltpu
from jax.experimental.pallas import tpu_sc as plsc

VOCAB_SIZE = 100000
EMBEDDING_SIZE = 64
BATCH = 4096
SEQ_LEN = 200

ROW = 100
ROWS_PER_SEQ = SEQ_LEN // ROW
NUM_WORKERS = 32
SEQS_PER_WORKER = BATCH // NUM_WORKERS


def _embed_kernel(idx_hbm, table_hbm, pe_hbm, out_hbm,
                  pe_v, idxa0, idxb0, idxa1, idxb1, rows0, rows1,
                  g0, g1, w0, w1):
    nc = 2
    wid = lax.axis_index("s") * nc + lax.axis_index("c")
    base_seq = wid * SEQS_PER_WORKER

    pltpu.sync_copy(pe_hbm, pe_v)

    idx_bufs = ((idxa0, idxb0), (idxa1, idxb1))
    row_bufs = (rows0, rows1)
    g_sems = (g0, g1)
    w_sems = (w0, w1)

    def fetch(s, b):
        ia, ib = idx_bufs[b]
        pltpu.sync_copy(idx_hbm.at[s, 0], ia)
        pltpu.sync_copy(idx_hbm.at[s, 1], ib)
        pltpu.async_copy(table_hbm.at[ia], row_bufs[b].at[0], g_sems[b])
        pltpu.async_copy(table_hbm.at[ib], row_bufs[b].at[1], g_sems[b])

    def gather_wait(b):
        ia, ib = idx_bufs[b]
        pltpu.make_async_copy(table_hbm.at[ia], row_bufs[b].at[0],
                              g_sems[b]).wait()
        pltpu.make_async_copy(table_hbm.at[ib], row_bufs[b].at[1],
                              g_sems[b]).wait()

    def wb_wait(b):
        pltpu.make_async_copy(row_bufs[b], out_hbm.at[base_seq],
                              w_sems[b]).wait()

    def add_pe(b):
        rows = row_bufs[b]

        def body(r, _):
            for a in range(ROWS_PER_SEQ):
                for c in range(EMBEDDING_SIZE // 16):
                    sl = pl.ds(c * 16, 16)
                    plsc.addupdate(rows.at[a, r, sl], pe_v[a, r, sl])
            return ()

        lax.fori_loop(0, ROW, body, (), unroll=4)

    fetch(base_seq, 0)

    def chunk_pair(p, _):
        s = base_seq + p * 2
        gather_wait(0)

        @pl.when(p > 0)
        def _():
            wb_wait(1)

        fetch(s + 1, 1)
        add_pe(0)
        pltpu.async_copy(row_bufs[0], out_hbm.at[s], w_sems[0])

        gather_wait(1)

        @pl.when(p + 1 < SEQS_PER_WORKER // 2)
        def _():
            wb_wait(0)
            fetch(s + 2, 0)

        add_pe(1)
        pltpu.async_copy(row_bufs[1], out_hbm.at[s + 1], w_sems[1])
        return ()

    lax.fori_loop(0, SEQS_PER_WORKER // 2, chunk_pair, ())

    wb_wait(0)
    wb_wait(1)


@jax.jit
def _run(idx_rows, table, pe_rows):
    mesh = plsc.VectorSubcoreMesh(core_axis_name="c", subcore_axis_name="s")
    fn = pl.kernel(
        _embed_kernel,
        mesh=mesh,
        compiler_params=pltpu.CompilerParams(use_tc_tiling_on_sc=False),
        out_type=jax.ShapeDtypeStruct(
            (BATCH, ROWS_PER_SEQ, ROW, EMBEDDING_SIZE), jnp.float32),
        scratch_types=[
            pltpu.VMEM((ROWS_PER_SEQ, ROW, EMBEDDING_SIZE), jnp.float32),
            pltpu.VMEM((ROW,), jnp.int32),
            pltpu.VMEM((ROW,), jnp.int32),
            pltpu.VMEM((ROW,), jnp.int32),
            pltpu.VMEM((ROW,), jnp.int32),
            pltpu.VMEM((ROWS_PER_SEQ, ROW, EMBEDDING_SIZE), jnp.float32),
            pltpu.VMEM((ROWS_PER_SEQ, ROW, EMBEDDING_SIZE), jnp.float32),
            pltpu.SemaphoreType.DMA,
            pltpu.SemaphoreType.DMA,
            pltpu.SemaphoreType.DMA,
            pltpu.SemaphoreType.DMA,
        ],
    )
    return fn(idx_rows, table, pe_rows)


def kernel(inputs, word_embedding, position_embedding):
    idx_rows = inputs.astype(jnp.int32).reshape(BATCH, ROWS_PER_SEQ, ROW)
    pe_rows = position_embedding[:SEQ_LEN].reshape(ROWS_PER_SEQ, ROW,
                                                   EMBEDDING_SIZE)
    out = _run(idx_rows, word_embedding, pe_rows)
    return out.reshape(BATCH, SEQ_LEN, EMBEDDING_SIZE)


# speedup vs baseline: 3.6618x; 1.1951x over previous
import jax
import jax.numpy as jnp
from jax import lax
from jax.experimental import pallas as pl
from jax.experimental.pallas import tpu as pltpu
from jax.experimental.pallas import tpu_sc as plsc

VOCAB_SIZE = 100000
EMBEDDING_SIZE = 64
BATCH = 4096
SEQ_LEN = 200

ROW = 100
ROWS_PER_SEQ = SEQ_LEN // ROW
NUM_WORKERS = 32
SEQS_PER_WORKER = BATCH // NUM_WORKERS
NBUF = 4


def _embed_kernel(idx_hbm, table_hbm, pe_hbm, out_hbm,
                  pe_v,
                  ia0, ib0, ia1, ib1, ia2, ib2, ia3, ib3,
                  rows0, rows1, rows2, rows3,
                  i0, i1, i2, i3, g0, g1, g2, g3, w0, w1, w2, w3):
    nc = 2
    wid = lax.axis_index("s") * nc + lax.axis_index("c")
    base_seq = wid * SEQS_PER_WORKER

    pltpu.sync_copy(pe_hbm, pe_v)

    idx_bufs = ((ia0, ib0), (ia1, ib1), (ia2, ib2), (ia3, ib3))
    row_bufs = (rows0, rows1, rows2, rows3)
    i_sems = (i0, i1, i2, i3)
    g_sems = (g0, g1, g2, g3)
    w_sems = (w0, w1, w2, w3)

    def idx_fetch(c, b):
        ia, ib = idx_bufs[b]
        s = base_seq + c
        pltpu.async_copy(idx_hbm.at[s, 0], ia, i_sems[b])
        pltpu.async_copy(idx_hbm.at[s, 1], ib, i_sems[b])

    def idx_wait(b):
        ia, ib = idx_bufs[b]
        pltpu.make_async_copy(idx_hbm.at[0, 0], ia, i_sems[b]).wait()
        pltpu.make_async_copy(idx_hbm.at[0, 0], ib, i_sems[b]).wait()

    def fetch(b):
        ia, ib = idx_bufs[b]
        pltpu.async_copy(table_hbm.at[ia], row_bufs[b].at[0], g_sems[b])
        pltpu.async_copy(table_hbm.at[ib], row_bufs[b].at[1], g_sems[b])

    def gather_wait(b):
        ia, ib = idx_bufs[b]
        pltpu.make_async_copy(table_hbm.at[ia], row_bufs[b].at[0],
                              g_sems[b]).wait()
        pltpu.make_async_copy(table_hbm.at[ib], row_bufs[b].at[1],
                              g_sems[b]).wait()

    def wb_wait(b):
        pltpu.make_async_copy(row_bufs[b], out_hbm.at[base_seq],
                              w_sems[b]).wait()

    def add_pe(b):
        rows = row_bufs[b]

        def body(r, _):
            for a in range(ROWS_PER_SEQ):
                for c in range(EMBEDDING_SIZE // 16):
                    sl = pl.ds(c * 16, 16)
                    plsc.addupdate(rows.at[a, r, sl], pe_v[a, r, sl])
            return ()

        lax.fori_loop(0, ROW, body, (), unroll=4)

    for b in range(NBUF):
        idx_fetch(b, b)
    for b in range(2):
        idx_wait(b)
        fetch(b)

    def outer(p, _):
        for b in range(NBUF):
            c = p * NBUF + b
            gather_wait(b)

            @pl.when(c + 4 < SEQS_PER_WORKER)
            def _():
                idx_fetch(c + 4, b)

            tb = (b + 2) % NBUF

            @pl.when(c >= 2)
            def _():
                wb_wait(tb)

            @pl.when(c + 2 < SEQS_PER_WORKER)
            def _():
                idx_wait(tb)
                fetch(tb)

            add_pe(b)
            pltpu.async_copy(row_bufs[b], out_hbm.at[base_seq + c],
                             w_sems[b])
        return ()

    lax.fori_loop(0, SEQS_PER_WORKER // NBUF, outer, ())

    wb_wait(2)
    wb_wait(3)


@jax.jit
def _run(idx_rows, table, pe_rows):
    mesh = plsc.VectorSubcoreMesh(core_axis_name="c", subcore_axis_name="s")
    fn = pl.kernel(
        _embed_kernel,
        mesh=mesh,
        compiler_params=pltpu.CompilerParams(use_tc_tiling_on_sc=False),
        out_type=jax.ShapeDtypeStruct(
            (BATCH, ROWS_PER_SEQ, ROW, EMBEDDING_SIZE), jnp.float32),
        scratch_types=[
            pltpu.VMEM((ROWS_PER_SEQ, ROW, EMBEDDING_SIZE), jnp.float32),
        ] + [
            pltpu.VMEM((ROW,), jnp.int32) for _ in range(2 * NBUF)
        ] + [
            pltpu.VMEM((ROWS_PER_SEQ, ROW, EMBEDDING_SIZE), jnp.float32)
            for _ in range(NBUF)
        ] + [pltpu.SemaphoreType.DMA for _ in range(3 * NBUF)],
    )
    return fn(idx_rows, table, pe_rows)


def kernel(inputs, word_embedding, position_embedding):
    idx_rows = inputs.astype(jnp.int32).reshape(BATCH, ROWS_PER_SEQ, ROW)
    pe_rows = position_embedding[:SEQ_LEN].reshape(ROWS_PER_SEQ, ROW,
                                                   EMBEDDING_SIZE)
    out = _run(idx_rows, word_embedding, pe_rows)
    return out.reshape(BATCH, SEQ_LEN, EMBEDDING_SIZE)


# speedup vs baseline: 4.2241x; 1.1536x over previous
import jax
import jax.numpy as jnp
from jax import lax
from jax.experimental import pallas as pl
from jax.experimental.pallas import tpu as pltpu
from jax.experimental.pallas import tpu_sc as plsc

VOCAB_SIZE = 100000
EMBEDDING_SIZE = 64
BATCH = 4096
SEQ_LEN = 200

NUM_WORKERS = 32
SEQS_PER_WORKER = BATCH // NUM_WORKERS
NBUF = 4
HALF = EMBEDDING_SIZE // 2


def _embed_kernel(idx_hbm, table_hbm, pe_hbm, out_hbm,
                  pe_v,
                  ix0, ix1, ix2, ix3,
                  rows0, rows1, rows2, rows3,
                  i0, i1, i2, i3, g0, g1, g2, g3, w0, w1, w2, w3):
    nc = 2
    wid = lax.axis_index("s") * nc + lax.axis_index("c")
    base_seq = wid * SEQS_PER_WORKER

    pltpu.sync_copy(pe_hbm, pe_v)

    idx_bufs = (ix0, ix1, ix2, ix3)
    row_bufs = (rows0, rows1, rows2, rows3)
    i_sems = (i0, i1, i2, i3)
    g_sems = (g0, g1, g2, g3)
    w_sems = (w0, w1, w2, w3)

    def idx_fetch(c, b):
        s = base_seq + c
        pltpu.async_copy(idx_hbm.at[pl.ds(s * SEQ_LEN, SEQ_LEN)],
                         idx_bufs[b], i_sems[b])

    def idx_wait(b):
        pltpu.make_async_copy(idx_hbm.at[pl.ds(0, SEQ_LEN)],
                              idx_bufs[b], i_sems[b]).wait()

    def fetch(b):
        pltpu.async_copy(table_hbm.at[idx_bufs[b]], row_bufs[b], g_sems[b])

    def gather_wait(b):
        pltpu.make_async_copy(table_hbm.at[idx_bufs[b]], row_bufs[b],
                              g_sems[b]).wait()

    def wb_wait(b):
        pltpu.make_async_copy(row_bufs[b], out_hbm.at[base_seq],
                              w_sems[b]).wait()

    def add_pe(b):
        rows = row_bufs[b]

        def body(r, _):
            for h in range(8):
                a = h // 4
                sl = pl.ds((h % 4) * 16, 16)
                plsc.addupdate(rows.at[2 * r + a, sl],
                               pe_v[r, pl.ds(h * 16, 16)])
            return ()

        lax.fori_loop(0, SEQ_LEN // 2, body, (), unroll=4)

    for b in range(NBUF):
        idx_fetch(b, b)
    for b in range(2):
        idx_wait(b)
        fetch(b)

    def outer(p, _):
        for b in range(NBUF):
            c = p * NBUF + b
            gather_wait(b)

            @pl.when(c + 4 < SEQS_PER_WORKER)
            def _():
                idx_fetch(c + 4, b)

            tb = (b + 2) % NBUF

            @pl.when(c >= 2)
            def _():
                wb_wait(tb)

            @pl.when(c + 2 < SEQS_PER_WORKER)
            def _():
                idx_wait(tb)
                fetch(tb)

            add_pe(b)
            pltpu.async_copy(row_bufs[b], out_hbm.at[base_seq + c],
                             w_sems[b])
        return ()

    lax.fori_loop(0, SEQS_PER_WORKER // NBUF, outer, ())

    wb_wait(2)
    wb_wait(3)


@jax.jit
def _run(idx_flat, table, pe_packed):
    mesh = plsc.VectorSubcoreMesh(core_axis_name="c", subcore_axis_name="s")
    fn = pl.kernel(
        _embed_kernel,
        mesh=mesh,
        compiler_params=pltpu.CompilerParams(use_tc_tiling_on_sc=False),
        out_type=jax.ShapeDtypeStruct((BATCH, SEQ_LEN, EMBEDDING_SIZE),
                                      jnp.float32),
        scratch_types=[
            pltpu.VMEM((SEQ_LEN // 2, 2 * EMBEDDING_SIZE), jnp.float32),
        ] + [
            pltpu.VMEM((SEQ_LEN,), jnp.int32) for _ in range(NBUF)
        ] + [
            pltpu.VMEM((SEQ_LEN, EMBEDDING_SIZE), jnp.float32)
            for _ in range(NBUF)
        ] + [pltpu.SemaphoreType.DMA for _ in range(3 * NBUF)],
    )
    return fn(idx_flat, table, pe_packed)


def kernel(inputs, word_embedding, position_embedding):
    idx_flat = inputs.astype(jnp.int32).reshape(BATCH * SEQ_LEN)
    pe_packed = position_embedding[:SEQ_LEN].reshape(SEQ_LEN // 2,
                                                     2 * EMBEDDING_SIZE)
    return _run(idx_flat, word_embedding, pe_packed)
